# baseline (speedup 1.0000x reference)
import functools
import jax
import jax.numpy as jnp
from jax import lax
from jax.experimental import pallas as pl
from jax.experimental.pallas import tpu as pltpu
from jax.experimental.pallas import tpu_sc as plsc

N_NODES = 10000
F_IN = 128
HID = 128
NCLASS = 16

NC = 2
NS = 16
FH = HID // NC
CHUNK = 128
SUP = 4
N_PAD = 10240
ROWS_PER_TILE = N_PAD // NS


def _mm_body(x_ref, w_ref, o_ref):
    r = jnp.dot(x_ref[...], w_ref[...], preferred_element_type=jnp.float32)
    o_ref[0] = r[:, :FH]
    o_ref[1] = r[:, FH:]


def _dense_matmul_split(x, w):
    n = x.shape[0]
    blk = 2000
    nb = n // blk
    out = pl.pallas_call(
        _mm_body,
        grid=(nb,),
        in_specs=[
            pl.BlockSpec((blk, F_IN), lambda i: (i, 0)),
            pl.BlockSpec((F_IN, HID), lambda i: (0, 0)),
        ],
        out_specs=pl.BlockSpec((NC, blk, FH), lambda i: (0, i, 0)),
        out_shape=jax.ShapeDtypeStruct((NC, n, FH), jnp.float32),
    )(x, w)
    return out.reshape(NC * n, FH)


def _spmm_body(h_hbm, src_hbm, dst_hbm, w_hbm, acc_out, c_out,
               msrc, mdst, mw, rows0, rows1, zbuf, acc_sh, c_sh,
               g0, g1, s0, s1, e0, e1, e2, e3, csem):
    cid = lax.axis_index("c")
    sid = lax.axis_index("s")
    n_super = dst_hbm.shape[1] - 2
    rows = (rows0, rows1)
    gsem = (g0, g1)
    ssem = (s0, s1)
    esem = (e0, e1, e2, e3)

    def zero_rows(buf):
        def zero_row(r, _):
            for f in range(FH // 16):
                buf[r, pl.ds(f * 16, 16)] = jnp.zeros((16,), jnp.float32)
            return _
        lax.fori_loop(0, SUP * CHUNK, zero_row, None)
    zero_rows(rows0)
    zero_rows(rows1)

    def zero_z(r, _):
        zbuf[pl.ds(r * 16, 16)] = jnp.zeros((16,), jnp.float32)
        return _
    lax.fori_loop(0, SUP * CHUNK // 16, zero_z, None)

    for t in range(ROWS_PER_TILE // CHUNK):
        off = sid * ROWS_PER_TILE + t * CHUNK
        pltpu.sync_copy(rows0.at[pl.ds(0, CHUNK)],
                        acc_sh.at[pl.ds(off, CHUNK)])
        pltpu.sync_copy(zbuf.at[pl.ds(0, CHUNK)],
                        c_sh.at[pl.ds(off, CHUNK)])
    plsc.subcore_barrier()

    def el_src(S, m, sem):
        return pltpu.make_async_copy(src_hbm.at[cid, sid, S], msrc.at[m], sem)

    def el_dst(S, m, sem):
        return pltpu.make_async_copy(dst_hbm.at[sid, S], mdst.at[m], sem)

    def el_w(S, m, sem):
        return pltpu.make_async_copy(w_hbm.at[sid, S], mw.at[m], sem)

    def eload_start(S, m):
        el_src(S, m, esem[m]).start()
        el_dst(S, m, esem[m]).start()
        el_w(S, m, esem[m]).start()

    def eload_wait(m):
        el_src(0, m, esem[m]).wait()
        el_dst(0, m, esem[m]).wait()
        el_w(0, m, esem[m]).wait()

    def gather(m, buf, sem):
        return pltpu.make_async_copy(h_hbm.at[msrc.at[m]], buf, sem)

    def scatter(m, buf, sem):
        return pltpu.make_async_copy(buf, acc_sh.at[mdst.at[m]], sem)

    def cscat(m):
        return pltpu.make_async_copy(mw.at[m], c_sh.at[msrc.at[m]], csem)

    def scale(buf, m):
        def scale_block(b, __):
            wvec = mw[m, pl.ds(b * 16, 16)]
            for l in range(16):
                i = b * 16 + l
                wb = jnp.full((16,), wvec[l], jnp.float32)
                for f in range(FH // 16):
                    sl = pl.ds(f * 16, 16)
                    buf[i, sl] = buf[i, sl] * wb
            return __
        lax.fori_loop(0, SUP * CHUNK // 16, scale_block, None)

    eload_start(0, 0)
    eload_start(1, 1)
    eload_wait(0)
    gather(0, rows0, g0).start()
    scatter(0, rows1, s1).start(add=True)

    @pl.when(cid == 0)
    def _():
        pltpu.make_async_copy(
            zbuf, c_sh.at[mdst.at[0]], csem).start(add=True)

    def step(S, k):
        p = k % 2
        q = 1 - p
        mn = (k + 1) % 4
        m2 = (k + 2) % 4
        eload_wait(mn)
        scatter(0, rows[q], ssem[q]).wait()
        gather(mn, rows[q], gsem[q]).start()
        gather(0, rows[p], gsem[p]).wait()
        eload_start(S + 2, m2)
        scale(rows[p], k)
        scatter(k, rows[p], ssem[p]).start(add=True)

        @pl.when(cid == 0)
        def _():
            cscat(0).wait()
            cscat(k).start(add=True)

    def pipe(s4, _):
        for k in range(4):
            step(s4 * 4 + k, k)
        return _

    lax.fori_loop(0, n_super // 4, pipe, None)
    gather(0, rows[0], gsem[0]).wait()
    scatter(0, rows[1], ssem[1]).wait()
    eload_wait(1)

    @pl.when(cid == 0)
    def _():
        cscat(0).wait()
    plsc.subcore_barrier()

    off = sid * ROWS_PER_TILE
    pltpu.sync_copy(acc_sh.at[pl.ds(off, ROWS_PER_TILE)],
                    acc_out.at[cid, pl.ds(off, ROWS_PER_TILE)])
    pltpu.sync_copy(c_sh.at[pl.ds(off, ROWS_PER_TILE)],
                    c_out.at[cid, pl.ds(off, ROWS_PER_TILE)])


def _spmm(h2, src5, dst4, w4):
    kern = functools.partial(
        pl.kernel,
        out_type=(
            jax.ShapeDtypeStruct((NC, N_PAD, FH), jnp.float32),
            jax.ShapeDtypeStruct((NC, N_PAD), jnp.float32),
        ),
        mesh=plsc.VectorSubcoreMesh(core_axis_name="c", subcore_axis_name="s"),
        compiler_params=pltpu.CompilerParams(use_tc_tiling_on_sc=False),
        scratch_types=[
            pltpu.VMEM((4, SUP * CHUNK), jnp.int32),
            pltpu.VMEM((4, SUP * CHUNK), jnp.int32),
            pltpu.VMEM((4, SUP * CHUNK), jnp.float32),
            pltpu.VMEM((SUP * CHUNK, FH), jnp.float32),
            pltpu.VMEM((SUP * CHUNK, FH), jnp.float32),
            pltpu.VMEM((SUP * CHUNK,), jnp.float32),
            pltpu.VMEM_SHARED((N_PAD, FH), jnp.float32),
            pltpu.VMEM_SHARED((N_PAD,), jnp.float32),
            pltpu.SemaphoreType.DMA,
            pltpu.SemaphoreType.DMA,
            pltpu.SemaphoreType.DMA,
            pltpu.SemaphoreType.DMA,
            pltpu.SemaphoreType.DMA,
            pltpu.SemaphoreType.DMA,
            pltpu.SemaphoreType.DMA,
            pltpu.SemaphoreType.DMA,
            pltpu.SemaphoreType.DMA,
        ],
    )(_spmm_body)
    return kern(h2, src5, dst4, w4)


def _reduce_body(a0_ref, a1_ref, c0_ref, c1_ref, w2a_ref, w2b_ref, o_ref):
    i = pl.program_id(0)
    cb = c0_ref[...] + c1_ref[...]
    s0 = jnp.sum(jnp.maximum(a0_ref[...], 0.0) * cb, axis=0)[None, :]
    s1 = jnp.sum(jnp.maximum(a1_ref[...], 0.0) * cb, axis=0)[None, :]
    val = (jnp.dot(s0, w2a_ref[...], preferred_element_type=jnp.float32)
           + jnp.dot(s1, w2b_ref[...], preferred_element_type=jnp.float32)
           ) * (1.0 / N_NODES)

    @pl.when(i == 0)
    def _():
        o_ref[...] = val

    @pl.when(i > 0)
    def _():
        o_ref[...] = o_ref[...] + val


def _reduce(acc, c, w2):
    blk = 1024
    grid = N_PAD // blk
    return pl.pallas_call(
        _reduce_body,
        grid=(grid,),
        in_specs=[
            pl.BlockSpec((blk, FH), lambda i: (i, 0)),
            pl.BlockSpec((blk, FH), lambda i: (i, 0)),
            pl.BlockSpec((blk, 1), lambda i: (i, 0)),
            pl.BlockSpec((blk, 1), lambda i: (i, 0)),
            pl.BlockSpec((FH, NCLASS), lambda i: (0, 0)),
            pl.BlockSpec((FH, NCLASS), lambda i: (0, 0)),
        ],
        out_specs=pl.BlockSpec((1, NCLASS), lambda i: (0, 0)),
        out_shape=jax.ShapeDtypeStruct((1, NCLASS), jnp.float32),
    )(acc[0], acc[1], c[0].reshape(N_PAD, 1), c[1].reshape(N_PAD, 1),
      w2[:FH], w2[FH:])


def kernel(x, edge_index, edge_weight, W1, W2):
    n = x.shape[0]
    e = edge_weight.shape[0]
    sup_edges = SUP * CHUNK
    per_tile = -(-e // (NS * 4 * sup_edges)) * 4 * sup_edges
    e_pad = per_tile * NS
    n_super = per_tile // sup_edges

    src = jnp.asarray(edge_index[0], jnp.int32)
    dst = jnp.asarray(edge_index[1], jnp.int32)
    w = jnp.asarray(edge_weight, jnp.float32)
    pad = e_pad - e
    src4 = jnp.pad(
        jnp.pad(src, (0, pad)).reshape(NS, n_super, SUP * CHUNK),
        ((0, 0), (0, 2), (0, 0)))
    dst4 = jnp.pad(
        jnp.pad(dst, (0, pad)).reshape(NS, n_super, SUP * CHUNK),
        ((0, 0), (0, 2), (0, 0)))
    w4 = jnp.pad(
        jnp.pad(w, (0, pad)).reshape(NS, n_super, SUP * CHUNK),
        ((0, 0), (0, 2), (0, 0)))
    src5 = jnp.stack([src4, src4 + n])

    h2 = _dense_matmul_split(x, W1)
    acc, c = _spmm(h2, src5, dst4, w4)
    return _reduce(acc, c, W2)
